# baseline (speedup 1.0000x reference)
import jax
import jax.numpy as jnp
from jax.experimental import pallas as pl
from jax.experimental.pallas import tpu as pltpu

_DM = 1024
_DI = 2048
_DS = 16
_DC = 4
_DR = 64
_B = 2
_L = 512
_TCH = 8
_NCH = _L // _TCH
_DBLK = 512
_NJ = _DI // _DBLK
_VMEM = 52 * 1024 * 1024

_DNT = (((1,), (1,)), ((), ()))


def _cp(sem):
    return pltpu.CompilerParams(dimension_semantics=sem,
                                vmem_limit_bytes=_VMEM)


def _dot_t(x, w):
    return jax.lax.dot_general(x, w, _DNT,
                               preferred_element_type=jnp.float32)


def _silu(v):
    return v * (1.0 / (1.0 + jnp.exp(-v)))


def _inproj_k(x_ref, w_ref, o_ref):
    o_ref[0] = _dot_t(x_ref[0], w_ref[...])


def _mid_k(xz_ref, cw_ref, cb_ref, wx_ref, wdt_ref, bdt_ref,
           u_ref, dlt_ref, bm_ref, cm_ref):
    u = xz_ref[0]
    acc = cb_ref[...] + cw_ref[_DC - 1:_DC, :] * u
    for s in range(1, _DC):
        ush = jnp.concatenate(
            [jnp.zeros((s, _DI), jnp.float32), u[:_L - s, :]], axis=0)
        acc = acc + cw_ref[_DC - 1 - s:_DC - s, :] * ush
    uc = _silu(acc)
    u_ref[0] = uc
    xdbl = _dot_t(uc, wx_ref[...])
    bm_ref[0] = xdbl[:, _DR:_DR + _DS]
    cm_ref[0] = xdbl[:, _DR + _DS:]
    pre = _dot_t(xdbl[:, :_DR], wdt_ref[...]) + bdt_ref[...]
    dlt_ref[0] = jnp.maximum(pre, 0.0) + jnp.log1p(jnp.exp(-jnp.abs(pre)))


def _scan_out_k(dlt_ref, u_ref, z_ref, bt_ref, cm_ref, alog_ref, d_ref,
                wo_ref, o_ref, y_scr):
    j = pl.program_id(1)
    neg_a = -jnp.exp(alog_ref[...])
    dp = d_ref[...]
    li = jax.lax.broadcasted_iota(jnp.int32, (_TCH, _TCH * _DS), 1)
    si = jax.lax.broadcasted_iota(jnp.int32, (_TCH, _TCH * _DS), 0)
    cmask = jnp.where((li // _DS) == si, 1.0, 0.0).astype(jnp.float32)

    def chunk(c, h):
        t0 = pl.multiple_of(c * _TCH, _TCH)
        d8 = dlt_ref[0, pl.ds(t0, _TCH), :]
        u8 = u_ref[0, pl.ds(t0, _TCH), :]
        g8 = _silu(z_ref[0, pl.ds(t0, _TCH), :])
        bt8 = bt_ref[0, c]
        cm8 = cm_ref[0, pl.ds(t0, _TCH), :]
        du8 = d8 * u8
        da8 = jnp.exp(d8.reshape(_TCH, 1, _DBLK) *
                      neg_a.reshape(1, _DS, _DBLK))
        hs = []
        for r in range(_TCH):
            bcol = jnp.broadcast_to(bt8[:, r:r + 1], (_DS, _DBLK))
            dub = jnp.broadcast_to(du8[r:r + 1, :], (_DS, _DBLK))
            h = da8[r] * h + bcol * dub
            hs.append(h)
        hstack = jnp.concatenate(hs, axis=0)
        csel = jnp.concatenate([cm8] * _TCH, axis=1) * cmask
        y8 = jax.lax.dot_general(csel, hstack, (((1,), (0,)), ((), ())),
                                 preferred_element_type=jnp.float32)
        y_scr[pl.ds(t0, _TCH), :] = (
            (y8 + u8 * jnp.broadcast_to(dp, (_TCH, _DBLK))) * g8)
        return h

    jax.lax.fori_loop(0, _NCH, chunk, jnp.zeros((_DS, _DBLK), jnp.float32))
    contrib = _dot_t(y_scr[...], wo_ref[...])

    @pl.when(j == 0)
    def _():
        o_ref[0] = contrib

    @pl.when(j > 0)
    def _():
        o_ref[0] = o_ref[0] + contrib


def _layer(x, wi, cw, cb, wx, wdt, bdt, alog, dvec, wo):
    f32 = jnp.float32
    xz = pl.pallas_call(
        _inproj_k,
        grid=(_B, 2 * _DI // 1024),
        in_specs=[
            pl.BlockSpec((1, _L, _DM), lambda b, j: (b, 0, 0)),
            pl.BlockSpec((1024, _DM), lambda b, j: (j, 0)),
        ],
        out_specs=pl.BlockSpec((1, _L, 1024), lambda b, j: (b, 0, j)),
        out_shape=jax.ShapeDtypeStruct((_B, _L, 2 * _DI), f32),
        compiler_params=_cp(("parallel", "parallel")),
    )(x, wi)

    u, dlt, bm, cm = pl.pallas_call(
        _mid_k,
        grid=(_B,),
        in_specs=[
            pl.BlockSpec((1, _L, _DI), lambda b: (b, 0, 0)),
            pl.BlockSpec((_DC, _DI), lambda b: (0, 0)),
            pl.BlockSpec((1, _DI), lambda b: (0, 0)),
            pl.BlockSpec((_DR + 2 * _DS, _DI), lambda b: (0, 0)),
            pl.BlockSpec((_DI, _DR), lambda b: (0, 0)),
            pl.BlockSpec((1, _DI), lambda b: (0, 0)),
        ],
        out_specs=[
            pl.BlockSpec((1, _L, _DI), lambda b: (b, 0, 0)),
            pl.BlockSpec((1, _L, _DI), lambda b: (b, 0, 0)),
            pl.BlockSpec((1, _L, _DS), lambda b: (b, 0, 0)),
            pl.BlockSpec((1, _L, _DS), lambda b: (b, 0, 0)),
        ],
        out_shape=[
            jax.ShapeDtypeStruct((_B, _L, _DI), f32),
            jax.ShapeDtypeStruct((_B, _L, _DI), f32),
            jax.ShapeDtypeStruct((_B, _L, _DS), f32),
            jax.ShapeDtypeStruct((_B, _L, _DS), f32),
        ],
        compiler_params=_cp(("parallel",)),
    )(xz, cw.T, cb.reshape(1, _DI), wx, wdt, bdt.reshape(1, _DI))

    bt = bm.reshape(_B, _NCH, _TCH, _DS).transpose(0, 1, 3, 2)

    return pl.pallas_call(
        _scan_out_k,
        grid=(_B, _NJ),
        in_specs=[
            pl.BlockSpec((1, _L, _DBLK), lambda b, j: (b, 0, j)),
            pl.BlockSpec((1, _L, _DBLK), lambda b, j: (b, 0, j)),
            pl.BlockSpec((1, _L, _DBLK), lambda b, j: (b, 0, _NJ + j)),
            pl.BlockSpec((1, _NCH, _DS, _TCH), lambda b, j: (b, 0, 0, 0)),
            pl.BlockSpec((1, _L, _DS), lambda b, j: (b, 0, 0)),
            pl.BlockSpec((_DS, _DBLK), lambda b, j: (0, j)),
            pl.BlockSpec((1, _DBLK), lambda b, j: (0, j)),
            pl.BlockSpec((_DM, _DBLK), lambda b, j: (0, j)),
        ],
        out_specs=pl.BlockSpec((1, _L, _DM), lambda b, j: (b, 0, 0)),
        out_shape=jax.ShapeDtypeStruct((_B, _L, _DM), f32),
        scratch_shapes=[pltpu.VMEM((_L, _DBLK), f32)],
        compiler_params=_cp(("parallel", "arbitrary")),
    )(dlt, u, xz, bt, cm, alog.T, dvec.reshape(1, _DI), wo)


def kernel(x, in_proj_w, conv_w, conv_b, x_proj_w, dt_proj_w, dt_proj_b,
           A_log, D, out_proj_w):
    for i in range(4):
        x = _layer(x, in_proj_w[i], conv_w[i], conv_b[i], x_proj_w[i],
                   dt_proj_w[i], dt_proj_b[i], A_log[i], D[i], out_proj_w[i])
    return x


# speedup vs baseline: 1.3883x; 1.3883x over previous
import jax
import jax.numpy as jnp
from jax.experimental import pallas as pl
from jax.experimental.pallas import tpu as pltpu

_DM = 1024
_DI = 2048
_DS = 16
_DC = 4
_DR = 64
_B = 2
_L = 512
_TCH = 8
_NCH = _L // _TCH
_DBLK = 512
_NJ = _DI // _DBLK
_VMEM = 52 * 1024 * 1024

_DNT = (((1,), (1,)), ((), ()))


def _cp(sem):
    return pltpu.CompilerParams(dimension_semantics=sem,
                                vmem_limit_bytes=_VMEM)


def _dot_t(x, w):
    return jax.lax.dot_general(x, w, _DNT,
                               preferred_element_type=jnp.float32)


def _silu(v):
    return v * (1.0 / (1.0 + jnp.exp(-v)))


def _inproj_k(x_ref, w_ref, o_ref):
    o_ref[0] = _dot_t(x_ref[0], w_ref[0])


def _mid_k(xz_ref, cw_ref, cb_ref, wx_ref, wdt_ref, bdt_ref,
           u_ref, dlt_ref, bm_ref, cm_ref):
    u = xz_ref[0]
    acc = cb_ref[0] + cw_ref[0, _DC - 1:_DC, :] * u
    for s in range(1, _DC):
        ush = jnp.concatenate(
            [jnp.zeros((s, _DI), jnp.float32), u[:_L - s, :]], axis=0)
        acc = acc + cw_ref[0, _DC - 1 - s:_DC - s, :] * ush
    uc = _silu(acc)
    u_ref[0] = uc
    xdbl = _dot_t(uc, wx_ref[0])
    bm_ref[0] = xdbl[:, _DR:_DR + _DS]
    cm_ref[0] = xdbl[:, _DR + _DS:]
    pre = _dot_t(xdbl[:, :_DR], wdt_ref[0]) + bdt_ref[0]
    dlt_ref[0] = jnp.maximum(pre, 0.0) + jnp.log1p(jnp.exp(-jnp.abs(pre)))


def _scan_out_k(dlt_ref, u_ref, z_ref, bct_ref, alog_ref, d_ref, wo_ref,
                o_ref, y_scr):
    j = pl.program_id(1)
    neg_a = -jnp.exp(alog_ref[0])
    dp = d_ref[0]

    def chunk(c, h):
        t0 = pl.multiple_of(c * _TCH, _TCH)
        d8 = dlt_ref[0, pl.ds(t0, _TCH), :]
        u8 = u_ref[0, pl.ds(t0, _TCH), :]
        g8 = _silu(z_ref[0, pl.ds(t0, _TCH), :])
        bc8 = bct_ref[0, c]
        du8 = d8 * u8
        da8 = jnp.exp(d8.reshape(_TCH, 1, _DBLK) *
                      neg_a.reshape(1, _DS, _DBLK))
        rows = []
        for r in range(_TCH):
            bcol = jnp.broadcast_to(bc8[0:_DS, r:r + 1], (_DS, _DBLK))
            ccol = jnp.broadcast_to(bc8[_DS:2 * _DS, r:r + 1], (_DS, _DBLK))
            dub = jnp.broadcast_to(du8[r:r + 1, :], (_DS, _DBLK))
            h = da8[r] * h + bcol * dub
            yr = jnp.sum(ccol * h, axis=0, keepdims=True)
            rows.append((yr + u8[r:r + 1, :] * dp) * g8[r:r + 1, :])
        y_scr[pl.ds(t0, _TCH), :] = jnp.concatenate(rows, axis=0)
        return h

    jax.lax.fori_loop(0, _NCH, chunk, jnp.zeros((_DS, _DBLK), jnp.float32))
    contrib = _dot_t(y_scr[...], wo_ref[0])

    @pl.when(j == 0)
    def _():
        o_ref[0] = contrib

    @pl.when(j > 0)
    def _():
        o_ref[0] = o_ref[0] + contrib


def _layer(i, x, wi_all, cwt_all, cb_all, wx_all, wdt_all, bdt_all,
           alogt_all, d_all, wo_all):
    f32 = jnp.float32
    xz = pl.pallas_call(
        _inproj_k,
        grid=(_B, 2 * _DI // 1024),
        in_specs=[
            pl.BlockSpec((1, _L, _DM), lambda b, j: (b, 0, 0)),
            pl.BlockSpec((1, 1024, _DM), lambda b, j: (i, j, 0)),
        ],
        out_specs=pl.BlockSpec((1, _L, 1024), lambda b, j: (b, 0, j)),
        out_shape=jax.ShapeDtypeStruct((_B, _L, 2 * _DI), f32),
        compiler_params=_cp(("parallel", "parallel")),
    )(x, wi_all)

    u, dlt, bm, cm = pl.pallas_call(
        _mid_k,
        grid=(_B,),
        in_specs=[
            pl.BlockSpec((1, _L, _DI), lambda b: (b, 0, 0)),
            pl.BlockSpec((1, _DC, _DI), lambda b: (i, 0, 0)),
            pl.BlockSpec((1, 1, _DI), lambda b: (i, 0, 0)),
            pl.BlockSpec((1, _DR + 2 * _DS, _DI), lambda b: (i, 0, 0)),
            pl.BlockSpec((1, _DI, _DR), lambda b: (i, 0, 0)),
            pl.BlockSpec((1, 1, _DI), lambda b: (i, 0, 0)),
        ],
        out_specs=[
            pl.BlockSpec((1, _L, _DI), lambda b: (b, 0, 0)),
            pl.BlockSpec((1, _L, _DI), lambda b: (b, 0, 0)),
            pl.BlockSpec((1, _L, _DS), lambda b: (b, 0, 0)),
            pl.BlockSpec((1, _L, _DS), lambda b: (b, 0, 0)),
        ],
        out_shape=[
            jax.ShapeDtypeStruct((_B, _L, _DI), f32),
            jax.ShapeDtypeStruct((_B, _L, _DI), f32),
            jax.ShapeDtypeStruct((_B, _L, _DS), f32),
            jax.ShapeDtypeStruct((_B, _L, _DS), f32),
        ],
        compiler_params=_cp(("parallel",)),
    )(xz, cwt_all, cb_all, wx_all, wdt_all, bdt_all)

    bct = jnp.concatenate(
        [bm.reshape(_B, _NCH, _TCH, _DS).transpose(0, 1, 3, 2),
         cm.reshape(_B, _NCH, _TCH, _DS).transpose(0, 1, 3, 2)], axis=2)

    return pl.pallas_call(
        _scan_out_k,
        grid=(_B, _NJ),
        in_specs=[
            pl.BlockSpec((1, _L, _DBLK), lambda b, j: (b, 0, j)),
            pl.BlockSpec((1, _L, _DBLK), lambda b, j: (b, 0, j)),
            pl.BlockSpec((1, _L, _DBLK), lambda b, j: (b, 0, _NJ + j)),
            pl.BlockSpec((1, _NCH, 2 * _DS, _TCH), lambda b, j: (b, 0, 0, 0)),
            pl.BlockSpec((1, _DS, _DBLK), lambda b, j: (i, 0, j)),
            pl.BlockSpec((1, 1, _DBLK), lambda b, j: (i, 0, j)),
            pl.BlockSpec((1, _DM, _DBLK), lambda b, j: (i, 0, j)),
        ],
        out_specs=pl.BlockSpec((1, _L, _DM), lambda b, j: (b, 0, 0)),
        out_shape=jax.ShapeDtypeStruct((_B, _L, _DM), f32),
        scratch_shapes=[pltpu.VMEM((_L, _DBLK), f32)],
        compiler_params=_cp(("parallel", "arbitrary")),
    )(dlt, u, xz, bct, alogt_all, d_all, wo_all)


def kernel(x, in_proj_w, conv_w, conv_b, x_proj_w, dt_proj_w, dt_proj_b,
           A_log, D, out_proj_w):
    cwt_all = conv_w.transpose(0, 2, 1)
    alogt_all = A_log.transpose(0, 2, 1)
    cb_all = conv_b.reshape(4, 1, _DI)
    bdt_all = dt_proj_b.reshape(4, 1, _DI)
    d_all = D.reshape(4, 1, _DI)
    for i in range(4):
        x = _layer(i, x, in_proj_w, cwt_all, cb_all, x_proj_w,
                   dt_proj_w, bdt_all, alogt_all, d_all, out_proj_w)
    return x
